# baseline (speedup 1.0000x reference)
import jax
import jax.numpy as jnp
from jax import lax
from jax.experimental import pallas as pl
from jax.experimental.pallas import tpu as pltpu

_BT = 2048


def _permute_matmul_kernel(idx_ref, x_ref, t_hbm, o_hbm, t_buf, o_buf,
                           t_sem, o_sem):
    P = t_hbm.shape[0]
    B, D = x_ref.shape
    nb = B // _BT
    nsteps = P * nb

    def t_copy(g):
        return pltpu.make_async_copy(
            t_hbm.at[idx_ref[g]], t_buf.at[g % 2], t_sem.at[g % 2])

    def o_copy(s):
        g, b = s // nb, s % nb
        return pltpu.make_async_copy(
            o_buf.at[s % 2],
            o_hbm.at[pl.ds((g * nb + b) * _BT, _BT), :],
            o_sem.at[s % 2],
        )

    t_copy(0).start()
    t_copy(1).start()
    for s in range(nsteps):
        g, b = s // nb, s % nb
        if b == 0:
            t_copy(g).wait()
        if s >= 2:
            o_copy(s - 2).wait()
        o_buf[s % 2] = lax.dot_general(
            x_ref[pl.ds(b * _BT, _BT), :],
            t_buf[g % 2],
            dimension_numbers=(((1,), (1,)), ((), ())),
            preferred_element_type=jnp.float32,
        )
        o_copy(s).start()
        if b == nb - 1 and g + 2 < P:
            t_copy(g + 2).start()
    o_copy(nsteps - 2).wait()
    o_copy(nsteps - 1).wait()


def kernel(x, T, indices):
    P, D, _ = T.shape
    B = x.shape[0]

    grid_spec = pltpu.PrefetchScalarGridSpec(
        num_scalar_prefetch=1,
        grid=(),
        in_specs=[
            pl.BlockSpec(memory_space=pltpu.VMEM),
            pl.BlockSpec(memory_space=pl.ANY),
        ],
        out_specs=pl.BlockSpec(memory_space=pl.ANY),
        scratch_shapes=[
            pltpu.VMEM((2, D, D), jnp.float32),
            pltpu.VMEM((2, _BT, D), jnp.float32),
            pltpu.SemaphoreType.DMA((2,)),
            pltpu.SemaphoreType.DMA((2,)),
        ],
    )
    return pl.pallas_call(
        _permute_matmul_kernel,
        out_shape=jax.ShapeDtypeStruct((P * B, D), jnp.float32),
        grid_spec=grid_spec,
        compiler_params=pltpu.CompilerParams(
            vmem_limit_bytes=56 * 1024 * 1024,
        ),
        name="permute_matmul",
    )(indices, x, T)


# speedup vs baseline: 1.0273x; 1.0273x over previous
import jax
import jax.numpy as jnp
from jax import lax
from jax.experimental import pallas as pl
from jax.experimental.pallas import tpu as pltpu

_BT = 2048


def _permute_matmul_kernel(idx_ref, x_ref, t_ref, o_ref):
    b = pl.program_id(1)
    row = pl.multiple_of(b * _BT, _BT)
    o_ref[...] = lax.dot_general(
        x_ref[pl.ds(row, _BT), :],
        t_ref[0],
        dimension_numbers=(((1,), (1,)), ((), ())),
        preferred_element_type=jnp.float32,
    )


def kernel(x, T, indices):
    P, D, _ = T.shape
    B = x.shape[0]
    nb = B // _BT

    grid_spec = pltpu.PrefetchScalarGridSpec(
        num_scalar_prefetch=1,
        grid=(P, nb),
        in_specs=[
            pl.BlockSpec((B, D), lambda g, b, idx: (0, 0)),
            pl.BlockSpec((1, D, D), lambda g, b, idx: (idx[g], 0, 0)),
        ],
        out_specs=pl.BlockSpec((_BT, D), lambda g, b, idx: (g * nb + b, 0)),
    )
    return pl.pallas_call(
        _permute_matmul_kernel,
        out_shape=jax.ShapeDtypeStruct((P * B, D), jnp.float32),
        grid_spec=grid_spec,
        compiler_params=pltpu.CompilerParams(
            dimension_semantics=("parallel", "arbitrary"),
            vmem_limit_bytes=56 * 1024 * 1024,
        ),
        name="permute_matmul",
    )(indices, x, T)
